# baseline (speedup 1.0000x reference)
import functools

import jax
import jax.numpy as jnp
from jax import lax
from jax.experimental import pallas as pl
from jax.experimental.pallas import tpu as pltpu
from jax.experimental.pallas import tpu_sc as plsc

D = 128
NP = 10240
NC, NS = 2, 16
NW = NC * NS
B = 80
ROWS_PER_SUB = NP // NS
CHUNK = 8


def _sc_agg_body(E, do_cnt, *refs):
    if do_cnt:
        (h_hbm, ea_hbm, src_hbm, dst_hbm, out_hbm, cnt_hbm,
         eidxA, eidxB, rowsA, rowsB, ones, cvec,
         agg_sh, cnt_sh, semA, semB) = refs
    else:
        (h_hbm, ea_hbm, src_hbm, dst_hbm, out_hbm,
         eidxA, eidxB, rowsA, rowsB,
         agg_sh, semA, semB) = refs
    c = lax.axis_index("c")
    s = lax.axis_index("s")
    epw = E // NW
    nblk = epw // B
    row0 = s * ROWS_PER_SUB
    base0 = (c * NS + s) * epw

    def zrow(r, _):
        for j in range(D // 16):
            rowsA[r, pl.ds(j * 16, 16)] = jnp.zeros((16,), jnp.float32)
        return 0
    lax.fori_loop(0, B, zrow, 0)
    for k in range(CHUNK):
        pltpu.sync_copy(rowsA, agg_sh.at[pl.ds(row0 + k * B, B)])
    if do_cnt:
        def zc(i, _):
            cvec[pl.ds(i * 16, 16)] = jnp.zeros((16,), jnp.float32)
            return 0
        lax.fori_loop(0, ROWS_PER_SUB // 16, zc, 0)
        for j in range(B // 16):
            ones[pl.ds(j * 16, 16)] = jnp.ones((16,), jnp.float32)
        pltpu.sync_copy(cvec, cnt_sh.at[pl.ds(row0, ROWS_PER_SUB)])
    plsc.subcore_barrier()

    def stage1(b, eidx, rows, sem):
        base = base0 + b * B
        pltpu.sync_copy(src_hbm.at[pl.ds(base, B)], eidx.at[0])
        pltpu.sync_copy(dst_hbm.at[pl.ds(base, B)], eidx.at[1])
        pltpu.async_copy(ea_hbm.at[pl.ds(base, B)], rows, sem)

    def stage2(b, eidx, rows, sem):
        base = base0 + b * B
        pltpu.make_async_copy(ea_hbm.at[pl.ds(base, B)], rows, sem).wait()
        pltpu.async_copy(h_hbm.at[eidx.at[0]], rows, sem, add=True)

    def process(b, eidx, rows, sem):
        pltpu.make_async_copy(h_hbm.at[eidx.at[0]], rows, sem).wait()

        def rowbody(r, _):
            for j in range(D // 16):
                sl = pl.ds(j * 16, 16)
                rows[r, sl] = jnp.maximum(rows[r, sl], 0.0)
            return 0
        lax.fori_loop(0, B, rowbody, 0)
        pltpu.sync_copy(rows, agg_sh.at[eidx.at[1]], add=True)
        if do_cnt:
            pltpu.sync_copy(ones, cnt_sh.at[eidx.at[1]], add=True)

    stage1(0, eidxA, rowsA, semA)
    stage1(1, eidxB, rowsB, semB)
    stage2(0, eidxA, rowsA, semA)

    def pair(i, _):
        g = 2 * i
        stage2(g + 1, eidxB, rowsB, semB)
        process(g, eidxA, rowsA, semA)
        stage1(g + 2, eidxA, rowsA, semA)
        process(g + 1, eidxB, rowsB, semB)
        stage2(g + 2, eidxA, rowsA, semA)
        stage1(jnp.minimum(g + 3, nblk - 1), eidxB, rowsB, semB)
        return 0
    lax.fori_loop(0, (nblk - 1) // 2, pair, 0)
    pltpu.make_async_copy(ea_hbm.at[pl.ds(base0 + (nblk - 1) * B, B)],
                          rowsB, semB).wait()
    process(nblk - 1, eidxA, rowsA, semA)
    plsc.subcore_barrier()

    for k in range(CHUNK):
        pltpu.sync_copy(agg_sh.at[pl.ds(row0 + k * B, B)],
                        out_hbm.at[pl.ds(c * NP + row0 + k * B, B)])
    if do_cnt:
        pltpu.sync_copy(cnt_sh.at[pl.ds(row0, ROWS_PER_SUB)],
                        cnt_hbm.at[pl.ds(c * NP + row0, ROWS_PER_SUB)])


@functools.lru_cache(maxsize=None)
def _make_sc_agg(E, do_cnt):
    mesh = plsc.VectorSubcoreMesh(core_axis_name="c", subcore_axis_name="s")
    if do_cnt:
        out_type = [jax.ShapeDtypeStruct((2 * NP, D), jnp.float32),
                    jax.ShapeDtypeStruct((2 * NP,), jnp.float32)]
    else:
        out_type = jax.ShapeDtypeStruct((2 * NP, D), jnp.float32)
    scratch = [
        pltpu.VMEM((2, B), jnp.int32),
        pltpu.VMEM((2, B), jnp.int32),
        pltpu.VMEM((B, D), jnp.float32),
        pltpu.VMEM((B, D), jnp.float32),
    ]
    if do_cnt:
        scratch += [
            pltpu.VMEM((B,), jnp.float32),
            pltpu.VMEM((ROWS_PER_SUB,), jnp.float32),
        ]
    scratch += [pltpu.VMEM_SHARED((NP, D), jnp.float32)]
    if do_cnt:
        scratch += [pltpu.VMEM_SHARED((NP,), jnp.float32)]
    scratch += [pltpu.SemaphoreType.DMA, pltpu.SemaphoreType.DMA]
    return pl.kernel(
        functools.partial(_sc_agg_body, E, do_cnt),
        out_type=out_type,
        mesh=mesh,
        scratch_types=scratch,
    )


EB = 2000


def _ea_body(a_ref, emb_ref, o_ref):
    o_ref[...] = lax.dot_general(
        a_ref[...], emb_ref[...], (((1,), (0,)), ((), ())),
        precision=lax.Precision.HIGHEST,
        preferred_element_type=jnp.float32)


@functools.lru_cache(maxsize=None)
def _make_ea(E, DE):
    return pl.pallas_call(
        _ea_body,
        grid=(E // EB,),
        in_specs=[
            pl.BlockSpec((EB, DE), lambda i: (i, 0)),
            pl.BlockSpec((DE, D), lambda i: (0, 0)),
        ],
        out_specs=pl.BlockSpec((EB, D), lambda i: (i, 0)),
        out_shape=jax.ShapeDtypeStruct((E, D), jnp.float32),
    )


RB = 1024


def _wr_body(h_ref, wr_ref, bl_ref, o_ref):
    z = lax.dot_general(h_ref[...], wr_ref[...], (((1,), (1,)), ((), ())),
                        precision=lax.Precision.HIGHEST,
                        preferred_element_type=jnp.float32)
    o_ref[...] = z + bl_ref[...]


@functools.lru_cache(maxsize=None)
def _make_wr():
    return pl.pallas_call(
        _wr_body,
        grid=(NP // RB,),
        in_specs=[
            pl.BlockSpec((RB, D), lambda i: (i, 0)),
            pl.BlockSpec((D, D), lambda i: (0, 0)),
            pl.BlockSpec((1, D), lambda i: (0, 0)),
        ],
        out_specs=pl.BlockSpec((RB, D), lambda i: (i, 0)),
        out_shape=jax.ShapeDtypeStruct((NP, D), jnp.float32),
    )


def _comb_body(apply_relu, p0_ref, p1_ref, c0_ref, c1_ref, hr_ref, wl_ref,
               o_ref):
    inv = 1.0 / jnp.maximum(c0_ref[...] + c1_ref[...], 1.0)
    agg = (p0_ref[...] + p1_ref[...]) * inv
    z = lax.dot_general(agg, wl_ref[...], (((1,), (1,)), ((), ())),
                        precision=lax.Precision.HIGHEST,
                        preferred_element_type=jnp.float32)
    z += hr_ref[...]
    o_ref[...] = jnp.maximum(z, 0.0) if apply_relu else z


@functools.lru_cache(maxsize=None)
def _make_comb(apply_relu):
    nb = NP // RB
    return pl.pallas_call(
        functools.partial(_comb_body, apply_relu),
        grid=(nb,),
        in_specs=[
            pl.BlockSpec((RB, D), lambda i: (i, 0)),
            pl.BlockSpec((RB, D), lambda i, nb=nb: (i + nb, 0)),
            pl.BlockSpec((RB, 1), lambda i: (i, 0)),
            pl.BlockSpec((RB, 1), lambda i, nb=nb: (i + nb, 0)),
            pl.BlockSpec((RB, D), lambda i: (i, 0)),
            pl.BlockSpec((D, D), lambda i: (0, 0)),
        ],
        out_specs=pl.BlockSpec((RB, D), lambda i: (i, 0)),
        out_shape=jax.ShapeDtypeStruct((NP, D), jnp.float32),
    )


def kernel(x, edge_index, edge_attr, emb_ea,
           Wl1, bl1, Wr1, Wl2, bl2, Wr2, Wl3, bl3, Wr3):
    N = x.shape[0]
    E = edge_index.shape[1]
    DE = edge_attr.shape[1]
    src = edge_index[0]
    dst = edge_index[1]

    ea = _make_ea(E, DE)(edge_attr, emb_ea)
    agg_cnt = _make_sc_agg(E, True)
    agg = _make_sc_agg(E, False)

    wr_call = _make_wr()
    h = jnp.zeros((NP, D), jnp.float32).at[:N].set(x)
    p, cnt = agg_cnt(h, ea, src, dst)
    hr = wr_call(h, Wr1, bl1.reshape(1, D))
    cnt = cnt.reshape(2 * NP, 1)
    h = _make_comb(True)(p, p, cnt, cnt, hr, Wl1)
    for Wl, bl, Wr, apply_relu in ((Wl2, bl2, Wr2, True),
                                   (Wl3, bl3, Wr3, False)):
        p = agg(h, ea, src, dst)
        hr = wr_call(h, Wr, bl.reshape(1, D))
        h = _make_comb(apply_relu)(p, p, cnt, cnt, hr, Wl)
    return h[:N]


# speedup vs baseline: 1.0548x; 1.0548x over previous
import functools

import jax
import jax.numpy as jnp
from jax import lax
from jax.experimental import pallas as pl
from jax.experimental.pallas import tpu as pltpu
from jax.experimental.pallas import tpu_sc as plsc

D = 128
NP = 10240
NC, NS = 2, 16
NW = NC * NS
B = 80
ROWS_PER_SUB = NP // NS
CHUNK = 8


def _sc_agg_body(E, do_cnt, *refs):
    if do_cnt:
        (h_hbm, ea_hbm, src_hbm, dst_hbm, out_hbm, cnt_hbm,
         eidxA, eidxB, rowsA, rowsB, eavA, eavB, ones, cvec,
         agg_sh, cnt_sh, semA, semB, semIA, semIB) = refs
    else:
        (h_hbm, ea_hbm, src_hbm, dst_hbm, out_hbm,
         eidxA, eidxB, rowsA, rowsB, eavA, eavB,
         agg_sh, semA, semB, semIA, semIB) = refs
    c = lax.axis_index("c")
    s = lax.axis_index("s")
    epw = E // NW
    nblk = epw // B
    row0 = s * ROWS_PER_SUB
    base0 = (c * NS + s) * epw

    def zrow(r, _):
        for j in range(D // 16):
            rowsA[r, pl.ds(j * 16, 16)] = jnp.zeros((16,), jnp.float32)
        return 0
    lax.fori_loop(0, B, zrow, 0)
    for k in range(CHUNK):
        pltpu.sync_copy(rowsA, agg_sh.at[pl.ds(row0 + k * B, B)])
    if do_cnt:
        def zc(i, _):
            cvec[pl.ds(i * 16, 16)] = jnp.zeros((16,), jnp.float32)
            return 0
        lax.fori_loop(0, ROWS_PER_SUB // 16, zc, 0)
        for j in range(B // 16):
            ones[pl.ds(j * 16, 16)] = jnp.ones((16,), jnp.float32)
        pltpu.sync_copy(cvec, cnt_sh.at[pl.ds(row0, ROWS_PER_SUB)])
    plsc.subcore_barrier()

    def issue_idx(b, eidx, semI):
        base = base0 + b * B
        pltpu.async_copy(src_hbm.at[pl.ds(base, B)], eidx.at[0], semI)
        pltpu.async_copy(dst_hbm.at[pl.ds(base, B)], eidx.at[1], semI)

    def issue_data(b, eidx, rows, eav, semI, sem):
        base = base0 + b * B
        pltpu.make_async_copy(src_hbm.at[pl.ds(base, B)], eidx.at[0],
                              semI).wait()
        pltpu.make_async_copy(dst_hbm.at[pl.ds(base, B)], eidx.at[1],
                              semI).wait()
        pltpu.async_copy(h_hbm.at[eidx.at[0]], rows, sem)
        pltpu.async_copy(ea_hbm.at[pl.ds(base, B)], eav, sem)

    def process(b, eidx, rows, eav, sem):
        base = base0 + b * B
        pltpu.make_async_copy(h_hbm.at[eidx.at[0]], rows, sem).wait()
        pltpu.make_async_copy(ea_hbm.at[pl.ds(base, B)], eav, sem).wait()

        def rowbody(r, _):
            for j in range(D // 16):
                sl = pl.ds(j * 16, 16)
                rows[r, sl] = jnp.maximum(rows[r, sl] + eav[r, sl], 0.0)
            return 0
        lax.fori_loop(0, B, rowbody, 0)
        pltpu.sync_copy(rows, agg_sh.at[eidx.at[1]], add=True)
        if do_cnt:
            pltpu.sync_copy(ones, cnt_sh.at[eidx.at[1]], add=True)

    issue_idx(0, eidxA, semIA)
    issue_idx(1, eidxB, semIB)
    issue_data(0, eidxA, rowsA, eavA, semIA, semA)

    def pair(i, _):
        g = 2 * i
        issue_data(g + 1, eidxB, rowsB, eavB, semIB, semB)
        process(g, eidxA, rowsA, eavA, semA)
        issue_idx(g + 2, eidxA, semIA)
        process(g + 1, eidxB, rowsB, eavB, semB)
        issue_data(g + 2, eidxA, rowsA, eavA, semIA, semA)
        issue_idx(jnp.minimum(g + 3, nblk - 1), eidxB, semIB)
        return 0
    lax.fori_loop(0, (nblk - 1) // 2, pair, 0)
    lastb = pl.ds(base0 + (nblk - 1) * B, B)
    pltpu.make_async_copy(src_hbm.at[lastb], eidxB.at[0], semIB).wait()
    pltpu.make_async_copy(dst_hbm.at[lastb], eidxB.at[1], semIB).wait()
    process(nblk - 1, eidxA, rowsA, eavA, semA)
    plsc.subcore_barrier()

    for k in range(CHUNK):
        pltpu.sync_copy(agg_sh.at[pl.ds(row0 + k * B, B)],
                        out_hbm.at[pl.ds(c * NP + row0 + k * B, B)])
    if do_cnt:
        pltpu.sync_copy(cnt_sh.at[pl.ds(row0, ROWS_PER_SUB)],
                        cnt_hbm.at[pl.ds(c * NP + row0, ROWS_PER_SUB)])


@functools.lru_cache(maxsize=None)
def _make_sc_agg(E, do_cnt):
    mesh = plsc.VectorSubcoreMesh(core_axis_name="c", subcore_axis_name="s")
    if do_cnt:
        out_type = [jax.ShapeDtypeStruct((2 * NP, D), jnp.float32),
                    jax.ShapeDtypeStruct((2 * NP,), jnp.float32)]
    else:
        out_type = jax.ShapeDtypeStruct((2 * NP, D), jnp.float32)
    scratch = [
        pltpu.VMEM((2, B), jnp.int32),
        pltpu.VMEM((2, B), jnp.int32),
        pltpu.VMEM((B, D), jnp.float32),
        pltpu.VMEM((B, D), jnp.float32),
        pltpu.VMEM((B, D), jnp.float32),
        pltpu.VMEM((B, D), jnp.float32),
    ]
    if do_cnt:
        scratch += [
            pltpu.VMEM((B,), jnp.float32),
            pltpu.VMEM((ROWS_PER_SUB,), jnp.float32),
        ]
    scratch += [pltpu.VMEM_SHARED((NP, D), jnp.float32)]
    if do_cnt:
        scratch += [pltpu.VMEM_SHARED((NP,), jnp.float32)]
    scratch += [pltpu.SemaphoreType.DMA, pltpu.SemaphoreType.DMA,
                pltpu.SemaphoreType.DMA, pltpu.SemaphoreType.DMA]
    return pl.kernel(
        functools.partial(_sc_agg_body, E, do_cnt),
        out_type=out_type,
        mesh=mesh,
        scratch_types=scratch,
    )


EB = 2000


def _ea_body(a_ref, emb_ref, o_ref):
    o_ref[...] = lax.dot_general(
        a_ref[...], emb_ref[...], (((1,), (0,)), ((), ())),
        precision=lax.Precision.HIGHEST,
        preferred_element_type=jnp.float32)


@functools.lru_cache(maxsize=None)
def _make_ea(E, DE):
    return pl.pallas_call(
        _ea_body,
        grid=(E // EB,),
        in_specs=[
            pl.BlockSpec((EB, DE), lambda i: (i, 0)),
            pl.BlockSpec((DE, D), lambda i: (0, 0)),
        ],
        out_specs=pl.BlockSpec((EB, D), lambda i: (i, 0)),
        out_shape=jax.ShapeDtypeStruct((E, D), jnp.float32),
    )


RB = 1024


def _wr_body(h_ref, wr_ref, bl_ref, o_ref):
    z = lax.dot_general(h_ref[...], wr_ref[...], (((1,), (1,)), ((), ())),
                        precision=lax.Precision.HIGHEST,
                        preferred_element_type=jnp.float32)
    o_ref[...] = z + bl_ref[...]


@functools.lru_cache(maxsize=None)
def _make_wr():
    return pl.pallas_call(
        _wr_body,
        grid=(NP // RB,),
        in_specs=[
            pl.BlockSpec((RB, D), lambda i: (i, 0)),
            pl.BlockSpec((D, D), lambda i: (0, 0)),
            pl.BlockSpec((1, D), lambda i: (0, 0)),
        ],
        out_specs=pl.BlockSpec((RB, D), lambda i: (i, 0)),
        out_shape=jax.ShapeDtypeStruct((NP, D), jnp.float32),
    )


def _comb_body(apply_relu, p0_ref, p1_ref, c0_ref, c1_ref, hr_ref, wl_ref,
               o_ref):
    inv = 1.0 / jnp.maximum(c0_ref[...] + c1_ref[...], 1.0)
    agg = (p0_ref[...] + p1_ref[...]) * inv
    z = lax.dot_general(agg, wl_ref[...], (((1,), (1,)), ((), ())),
                        precision=lax.Precision.HIGHEST,
                        preferred_element_type=jnp.float32)
    z += hr_ref[...]
    o_ref[...] = jnp.maximum(z, 0.0) if apply_relu else z


@functools.lru_cache(maxsize=None)
def _make_comb(apply_relu):
    nb = NP // RB
    return pl.pallas_call(
        functools.partial(_comb_body, apply_relu),
        grid=(nb,),
        in_specs=[
            pl.BlockSpec((RB, D), lambda i: (i, 0)),
            pl.BlockSpec((RB, D), lambda i, nb=nb: (i + nb, 0)),
            pl.BlockSpec((RB, 1), lambda i: (i, 0)),
            pl.BlockSpec((RB, 1), lambda i, nb=nb: (i + nb, 0)),
            pl.BlockSpec((RB, D), lambda i: (i, 0)),
            pl.BlockSpec((D, D), lambda i: (0, 0)),
        ],
        out_specs=pl.BlockSpec((RB, D), lambda i: (i, 0)),
        out_shape=jax.ShapeDtypeStruct((NP, D), jnp.float32),
    )


def kernel(x, edge_index, edge_attr, emb_ea,
           Wl1, bl1, Wr1, Wl2, bl2, Wr2, Wl3, bl3, Wr3):
    N = x.shape[0]
    E = edge_index.shape[1]
    DE = edge_attr.shape[1]

    ea = _make_ea(E, DE)(edge_attr, emb_ea)
    agg_cnt = _make_sc_agg(E, True)
    agg = _make_sc_agg(E, False)

    wr_call = _make_wr()
    h = jnp.zeros((NP, D), jnp.float32).at[:N].set(x)
    src = edge_index[0]
    dst = edge_index[1]
    p, cnt = agg_cnt(h, ea, src, dst)
    hr = wr_call(h, Wr1, bl1.reshape(1, D))
    cnt = cnt.reshape(2 * NP, 1)
    h = _make_comb(True)(p, p, cnt, cnt, hr, Wl1)
    for Wl, bl, Wr, apply_relu in ((Wl2, bl2, Wr2, True),
                                   (Wl3, bl3, Wr3, False)):
        p = agg(h, ea, src, dst)
        hr = wr_call(h, Wr, bl.reshape(1, D))
        h = _make_comb(apply_relu)(p, p, cnt, cnt, hr, Wl)
    return h[:N]


# speedup vs baseline: 1.0845x; 1.0281x over previous
import functools

import jax
import jax.numpy as jnp
from jax import lax
from jax.experimental import pallas as pl
from jax.experimental.pallas import tpu as pltpu
from jax.experimental.pallas import tpu_sc as plsc

D = 128
NP = 10240
NC, NS = 2, 16
NW = NC * NS
B = 80
ROWS_PER_SUB = NP // NS
CHUNK = 8


def _sc_agg_body(E, do_cnt, *refs):
    if do_cnt:
        (h_hbm, ea_hbm, src_hbm, dst_hbm, out_hbm, cnt_hbm,
         eidxA, eidxB, rowsA, rowsB, eavA, eavB, ones, cvec,
         agg_sh, cnt_sh, semA, semB) = refs
    else:
        (h_hbm, ea_hbm, src_hbm, dst_hbm, out_hbm,
         eidxA, eidxB, rowsA, rowsB, eavA, eavB,
         agg_sh, semA, semB) = refs
    c = lax.axis_index("c")
    s = lax.axis_index("s")
    epw = E // NW
    nblk = epw // B
    row0 = s * ROWS_PER_SUB
    base0 = (c * NS + s) * epw

    def zrow(r, _):
        for j in range(D // 16):
            rowsA[r, pl.ds(j * 16, 16)] = jnp.zeros((16,), jnp.float32)
        return 0
    lax.fori_loop(0, B, zrow, 0)
    for k in range(CHUNK):
        pltpu.sync_copy(rowsA, agg_sh.at[pl.ds(row0 + k * B, B)])
    if do_cnt:
        def zc(i, _):
            cvec[pl.ds(i * 16, 16)] = jnp.zeros((16,), jnp.float32)
            return 0
        lax.fori_loop(0, ROWS_PER_SUB // 16, zc, 0)
        for j in range(B // 16):
            ones[pl.ds(j * 16, 16)] = jnp.ones((16,), jnp.float32)
        pltpu.sync_copy(cvec, cnt_sh.at[pl.ds(row0, ROWS_PER_SUB)])
    plsc.subcore_barrier()

    def issue(b, eidx, rows, eav, sem):
        base = base0 + b * B
        pltpu.sync_copy(src_hbm.at[pl.ds(base, B)], eidx.at[0])
        pltpu.sync_copy(dst_hbm.at[pl.ds(base, B)], eidx.at[1])
        pltpu.async_copy(h_hbm.at[eidx.at[0]], rows, sem)
        pltpu.async_copy(ea_hbm.at[pl.ds(base, B)], eav, sem)

    def process(b, eidx, rows, eav, sem):
        base = base0 + b * B
        pltpu.make_async_copy(h_hbm.at[eidx.at[0]], rows, sem).wait()
        pltpu.make_async_copy(ea_hbm.at[pl.ds(base, B)], eav, sem).wait()

        def rowbody(r, _):
            for j in range(D // 16):
                sl = pl.ds(j * 16, 16)
                rows[r, sl] = jnp.maximum(rows[r, sl] + eav[r, sl], 0.0)
            return 0
        lax.fori_loop(0, B, rowbody, 0)
        pltpu.sync_copy(rows, agg_sh.at[eidx.at[1]], add=True)
        if do_cnt:
            pltpu.sync_copy(ones, cnt_sh.at[eidx.at[1]], add=True)

    issue(0, eidxA, rowsA, eavA, semA)

    def pair(i, _):
        g = 2 * i
        issue(g + 1, eidxB, rowsB, eavB, semB)
        process(g, eidxA, rowsA, eavA, semA)
        issue(g + 2, eidxA, rowsA, eavA, semA)
        process(g + 1, eidxB, rowsB, eavB, semB)
        return 0
    lax.fori_loop(0, (nblk - 1) // 2, pair, 0)
    process(nblk - 1, eidxA, rowsA, eavA, semA)
    plsc.subcore_barrier()

    for k in range(CHUNK):
        pltpu.sync_copy(agg_sh.at[pl.ds(row0 + k * B, B)],
                        out_hbm.at[pl.ds(c * NP + row0 + k * B, B)])
    if do_cnt:
        pltpu.sync_copy(cnt_sh.at[pl.ds(row0, ROWS_PER_SUB)],
                        cnt_hbm.at[pl.ds(c * NP + row0, ROWS_PER_SUB)])


@functools.lru_cache(maxsize=None)
def _make_sc_agg(E, do_cnt):
    mesh = plsc.VectorSubcoreMesh(core_axis_name="c", subcore_axis_name="s")
    if do_cnt:
        out_type = [jax.ShapeDtypeStruct((2 * NP, D), jnp.float32),
                    jax.ShapeDtypeStruct((2 * NP,), jnp.float32)]
    else:
        out_type = jax.ShapeDtypeStruct((2 * NP, D), jnp.float32)
    scratch = [
        pltpu.VMEM((2, B), jnp.int32),
        pltpu.VMEM((2, B), jnp.int32),
        pltpu.VMEM((B, D), jnp.float32),
        pltpu.VMEM((B, D), jnp.float32),
        pltpu.VMEM((B, D), jnp.float32),
        pltpu.VMEM((B, D), jnp.float32),
    ]
    if do_cnt:
        scratch += [
            pltpu.VMEM((B,), jnp.float32),
            pltpu.VMEM((ROWS_PER_SUB,), jnp.float32),
        ]
    scratch += [pltpu.VMEM_SHARED((NP, D), jnp.float32)]
    if do_cnt:
        scratch += [pltpu.VMEM_SHARED((NP,), jnp.float32)]
    scratch += [pltpu.SemaphoreType.DMA, pltpu.SemaphoreType.DMA]
    return pl.kernel(
        functools.partial(_sc_agg_body, E, do_cnt),
        out_type=out_type,
        mesh=mesh,
        scratch_types=scratch,
    )


EB = 2000


def _ea_body(a_ref, emb_ref, o_ref):
    o_ref[...] = lax.dot_general(
        a_ref[...], emb_ref[...], (((1,), (0,)), ((), ())),
        precision=lax.Precision.HIGHEST,
        preferred_element_type=jnp.float32)


@functools.lru_cache(maxsize=None)
def _make_ea(E, DE):
    return pl.pallas_call(
        _ea_body,
        grid=(E // EB,),
        in_specs=[
            pl.BlockSpec((EB, DE), lambda i: (i, 0)),
            pl.BlockSpec((DE, D), lambda i: (0, 0)),
        ],
        out_specs=pl.BlockSpec((EB, D), lambda i: (i, 0)),
        out_shape=jax.ShapeDtypeStruct((E, D), jnp.float32),
    )


RB = 1024


def _wr_body(h_ref, wr_ref, bl_ref, o_ref):
    z = lax.dot_general(h_ref[...], wr_ref[...], (((1,), (1,)), ((), ())),
                        precision=lax.Precision.HIGHEST,
                        preferred_element_type=jnp.float32)
    o_ref[...] = z + bl_ref[...]


@functools.lru_cache(maxsize=None)
def _make_wr():
    return pl.pallas_call(
        _wr_body,
        grid=(NP // RB,),
        in_specs=[
            pl.BlockSpec((RB, D), lambda i: (i, 0)),
            pl.BlockSpec((D, D), lambda i: (0, 0)),
            pl.BlockSpec((1, D), lambda i: (0, 0)),
        ],
        out_specs=pl.BlockSpec((RB, D), lambda i: (i, 0)),
        out_shape=jax.ShapeDtypeStruct((NP, D), jnp.float32),
    )


def _comb_body(apply_relu, p0_ref, p1_ref, c0_ref, c1_ref, hr_ref, wl_ref,
               o_ref):
    inv = 1.0 / jnp.maximum(c0_ref[...] + c1_ref[...], 1.0)
    agg = (p0_ref[...] + p1_ref[...]) * inv
    z = lax.dot_general(agg, wl_ref[...], (((1,), (1,)), ((), ())),
                        precision=lax.Precision.HIGHEST,
                        preferred_element_type=jnp.float32)
    z += hr_ref[...]
    o_ref[...] = jnp.maximum(z, 0.0) if apply_relu else z


@functools.lru_cache(maxsize=None)
def _make_comb(apply_relu):
    nb = NP // RB
    return pl.pallas_call(
        functools.partial(_comb_body, apply_relu),
        grid=(nb,),
        in_specs=[
            pl.BlockSpec((RB, D), lambda i: (i, 0)),
            pl.BlockSpec((RB, D), lambda i, nb=nb: (i + nb, 0)),
            pl.BlockSpec((RB, 1), lambda i: (i, 0)),
            pl.BlockSpec((RB, 1), lambda i, nb=nb: (i + nb, 0)),
            pl.BlockSpec((RB, D), lambda i: (i, 0)),
            pl.BlockSpec((D, D), lambda i: (0, 0)),
        ],
        out_specs=pl.BlockSpec((RB, D), lambda i: (i, 0)),
        out_shape=jax.ShapeDtypeStruct((NP, D), jnp.float32),
    )


def kernel(x, edge_index, edge_attr, emb_ea,
           Wl1, bl1, Wr1, Wl2, bl2, Wr2, Wl3, bl3, Wr3):
    N = x.shape[0]
    E = edge_index.shape[1]
    DE = edge_attr.shape[1]

    ea = _make_ea(E, DE)(edge_attr, emb_ea)
    agg_cnt = _make_sc_agg(E, True)
    agg = _make_sc_agg(E, False)

    wr_call = _make_wr()
    h = jnp.zeros((NP, D), jnp.float32).at[:N].set(x)
    src = edge_index[0]
    dst = edge_index[1]
    p, cnt = agg_cnt(h, ea, src, dst)
    hr = wr_call(h, Wr1, bl1.reshape(1, D))
    cnt = cnt.reshape(2 * NP, 1)
    h = _make_comb(True)(p, p, cnt, cnt, hr, Wl1)
    for Wl, bl, Wr, apply_relu in ((Wl2, bl2, Wr2, True),
                                   (Wl3, bl3, Wr3, False)):
        p = agg(h, ea, src, dst)
        hr = wr_call(h, Wr, bl.reshape(1, D))
        h = _make_comb(apply_relu)(p, p, cnt, cnt, hr, Wl)
    return h[:N]
